# baseline (speedup 1.0000x reference)
import functools

import jax
import jax.numpy as jnp
from jax import lax
from jax.experimental import pallas as pl
from jax.experimental.pallas import tpu as pltpu
from jax.experimental.pallas import tpu_sc as plsc

N = 10000
E = 320000
D = 128
NP = 10240
RPT = NP // 16
NW = 32
EPW = E // NW
K = 80
NCHUNK = EPW // K


def _mm_body(x_ref, w_ref, a2_ref, h_ref, s_ref):
    h = jnp.dot(x_ref[...], w_ref[...], preferred_element_type=jnp.float32)
    h_ref[...] = h
    s_ref[...] = lax.dot_general(
        a2_ref[...], h, (((0,), (1,)), ((), ())),
        preferred_element_type=jnp.float32,
    )


def _phase_a(x, W, A2):
    BN = 512
    grid = NP // BN
    return pl.pallas_call(
        _mm_body,
        grid=(grid,),
        in_specs=[
            pl.BlockSpec((BN, D), lambda i: (i, 0)),
            pl.BlockSpec((D, D), lambda i: (0, 0)),
            pl.BlockSpec((D, D), lambda i: (0, 0)),
        ],
        out_specs=[
            pl.BlockSpec((BN, D), lambda i: (i, 0)),
            pl.BlockSpec((D, BN), lambda i: (0, i)),
        ],
        out_shape=[
            jax.ShapeDtypeStruct((N, D), jnp.float32),
            jax.ShapeDtypeStruct((D, NP), jnp.float32),
        ],
    )(x, W, A2)


_MESH = plsc.VectorSubcoreMesh(
    core_axis_name="c", subcore_axis_name="s", num_cores=2, num_subcores=16
)


@functools.partial(
    pl.kernel,
    out_type=[
        jax.ShapeDtypeStruct((NP, D), jnp.float32),
        jax.ShapeDtypeStruct((NP, D), jnp.float32),
        jax.ShapeDtypeStruct((NP,), jnp.float32),
        jax.ShapeDtypeStruct((NP,), jnp.float32),
    ],
    mesh=_MESH,
    compiler_params=pltpu.CompilerParams(
        needs_layout_passes=False, use_tc_tiling_on_sc=False
    ),
    scratch_types=[
        [pltpu.VMEM((4 * K,), jnp.int32)] * 2,
        [pltpu.VMEM((4 * K,), jnp.int32)] * 2,
        [pltpu.VMEM((K,), jnp.float32)] * 4,
        [pltpu.VMEM((K,), jnp.float32)] * 4,
        [pltpu.VMEM((K,), jnp.int32)] * 4,
        [pltpu.VMEM((K,), jnp.float32)] * 4,
        [pltpu.VMEM((K, D), jnp.float32)] * 4,
        pltpu.VMEM_SHARED((NP, D), jnp.float32),
        pltpu.VMEM_SHARED((NP,), jnp.float32),
        pltpu.SemaphoreType.DMA,
        pltpu.SemaphoreType.DMA,
        pltpu.SemaphoreType.DMA,
    ],
)
def _phase_b(src_hbm, dst_hbm, s1_hbm, s2_hbm, h_hbm,
             u0, u1, den0, den1,
             qsrc, qdst, s1g, s2g, scb, pbuf, rows, acc, dacc,
             gsem, isem, ssem):
    cid = lax.axis_index("c")
    sid = lax.axis_index("s")
    wid = sid * 2 + cid
    r0 = sid * RPT
    base0 = wid * EPW
    zf = jnp.zeros((16,), jnp.float32)

    def _zrow(i, _):
        for j in range(D // 16):
            rows[0][i, pl.ds(j * 16, 16)] = zf
        return 0
    lax.fori_loop(0, K, _zrow, 0)
    for i in range(K // 16):
        pbuf[0][pl.ds(i * 16, 16)] = zf
    for c in range(RPT // K):
        pltpu.async_copy(rows[0], acc.at[pl.ds(r0 + c * K, K)], ssem)
        pltpu.async_copy(pbuf[0], dacc.at[pl.ds(r0 + c * K, K)], ssem)
    for c in range(RPT // K):
        pltpu.make_async_copy(rows[0], acc.at[pl.ds(r0 + c * K, K)], ssem).wait()
        pltpu.make_async_copy(pbuf[0], dacc.at[pl.ds(r0 + c * K, K)], ssem).wait()
    plsc.subcore_barrier()

    QB = 4 * K

    def _quad_start(G, slot, n):
        e0 = base0 + G * QB
        pltpu.async_copy(src_hbm.at[pl.ds(e0, n)], qsrc[slot].at[pl.ds(0, n)],
                         isem)
        pltpu.async_copy(dst_hbm.at[pl.ds(e0, n)], qdst[slot].at[pl.ds(0, n)],
                         isem)

    def _quad_wait(slot, n):
        pltpu.make_async_copy(src_hbm.at[pl.ds(0, n)],
                              qsrc[slot].at[pl.ds(0, n)], isem).wait()
        pltpu.make_async_copy(dst_hbm.at[pl.ds(0, n)],
                              qdst[slot].at[pl.ds(0, n)], isem).wait()

    def _gather_start(q, pos, r):
        pltpu.async_copy(h_hbm.at[qdst[q].at[pl.ds(pos * K, K)]], rows[r],
                         gsem)
        pltpu.async_copy(s1_hbm.at[qsrc[q].at[pl.ds(pos * K, K)]], s1g[r],
                         gsem)
        pltpu.async_copy(s2_hbm.at[qdst[q].at[pl.ds(pos * K, K)]], s2g[r],
                         gsem)

    def _gather_wait(r):
        pltpu.make_async_copy(h_hbm.at[qdst[0].at[pl.ds(0, K)]], rows[r],
                              gsem).wait()
        pltpu.make_async_copy(s1_hbm.at[qsrc[0].at[pl.ds(0, K)]], s1g[r],
                              gsem).wait()
        pltpu.make_async_copy(s2_hbm.at[qdst[0].at[pl.ds(0, K)]], s2g[r],
                              gsem).wait()

    def _scatter_start(r):
        pltpu.async_copy(rows[r], acc.at[scb[r]], ssem, add=True)
        pltpu.async_copy(pbuf[r], dacc.at[scb[r]], ssem, add=True)

    def _scatter_wait(r):
        pltpu.make_async_copy(rows[r], acc.at[scb[r]], ssem).wait()
        pltpu.make_async_copy(pbuf[r], dacc.at[scb[r]], ssem).wait()

    def _p_loop(q, pos, r):
        for i in range(K // 16):
            isrc = qsrc[q][pl.ds(pos * K + i * 16, 16)]
            scb[r][pl.ds(i * 16, 16)] = isrc
            v = s1g[r][pl.ds(i * 16, 16)] + s2g[r][pl.ds(i * 16, 16)]
            e = jnp.where(v > 0, v, 0.2 * v)
            pbuf[r][pl.ds(i * 16, 16)] = jnp.exp(e)

    def _scale(r):
        def body(i2, _):
            for v in range(4):
                i = i2 * 4 + v
                bc = plsc.load_gather(pbuf[r], [lax.broadcast(i, (16,))])
                for j in range(D // 16):
                    rows[r][i, pl.ds(j * 16, 16)] = (
                        rows[r][i, pl.ds(j * 16, 16)] * bc
                    )
            return 0
        lax.fori_loop(0, K // 4, body, 0)

    pltpu.sync_copy(src_hbm.at[pl.ds(base0, QB)], qsrc[0])
    pltpu.sync_copy(dst_hbm.at[pl.ds(base0, QB)], qdst[0])
    _gather_start(0, 0, 0)
    _gather_start(0, 1, 1)
    _quad_start(1, 1, QB)

    def _iter(G, q, c, uu, last_quad):
        t = 4 * G + c
        _gather_wait(c)
        _p_loop(q, c, c)

        @pl.when(t >= 2)
        def _():
            _scatter_wait((c + 2) % 4)

        if c == 2 and not last_quad:
            _quad_wait(1 - q, QB)
        if c == 2 and last_quad:
            _quad_wait(1 - q, K)

        if not (last_quad and c == 3):
            if c <= 1:
                _gather_start(q, c + 2, (c + 2) % 4)
            else:
                _gather_start(1 - q, c - 2, (c + 2) % 4)

        if c == 3 and uu is not None:
            if q == 0:
                _quad_start(2 * uu + 2, 0, QB)
            else:
                @pl.when(uu < 14)
                def _():
                    _quad_start(2 * uu + 3, 1, QB)

                @pl.when(uu == 14)
                def _():
                    _quad_start(2 * uu + 3, 1, K)

        _scale(c)
        _scatter_start(c)

    def _outer(uu, _):
        for P in (0, 1):
            for c in range(4):
                _iter(2 * uu + P, P, c, uu, False)
        return 0
    lax.fori_loop(0, 15, _outer, 0)

    for c in range(4):
        _iter(30, 0, c, None, True)

    _gather_wait(0)
    _scatter_wait(2)
    _p_loop(1, 0, 0)
    _scale(0)
    _scatter_start(0)
    _scatter_wait(3)
    _scatter_wait(0)
    plsc.subcore_barrier()

    @pl.when(cid == 0)
    def _():
        pltpu.sync_copy(acc.at[pl.ds(r0, RPT)], u0.at[pl.ds(r0, RPT)])
        pltpu.sync_copy(dacc.at[pl.ds(r0, RPT)], den0.at[pl.ds(r0, RPT)])

    @pl.when(cid == 1)
    def _():
        pltpu.sync_copy(acc.at[pl.ds(r0, RPT)], u1.at[pl.ds(r0, RPT)])
        pltpu.sync_copy(dacc.at[pl.ds(r0, RPT)], den1.at[pl.ds(r0, RPT)])


def _fin_body(u0_ref, u1_ref, d0_ref, d1_ref, o_ref):
    u = u0_ref[...] + u1_ref[...]
    d = d0_ref[...] + d1_ref[...]
    r = jnp.where(d > 0, 1.0 / jnp.where(d > 0, d, 1.0), 0.0)
    hp = u * r[:, None]
    o_ref[...] = jnp.where(hp > 0, hp, jnp.exp(jnp.minimum(hp, 0.0)) - 1.0)


def _phase_c(u0, u1, d0, d1):
    BN = 512
    grid = NP // BN
    return pl.pallas_call(
        _fin_body,
        grid=(grid,),
        in_specs=[
            pl.BlockSpec((BN, D), lambda i: (i, 0)),
            pl.BlockSpec((BN, D), lambda i: (i, 0)),
            pl.BlockSpec((BN,), lambda i: (i,)),
            pl.BlockSpec((BN,), lambda i: (i,)),
        ],
        out_specs=pl.BlockSpec((BN, D), lambda i: (i, 0)),
        out_shape=jax.ShapeDtypeStruct((N, D), jnp.float32),
    )(u0, u1, d0, d1)


def kernel(input, edge_list, W, a):
    A2 = jnp.zeros((D, D), jnp.float32)
    A2 = A2.at[:, 0].set(a[:D, 0]).at[:, 1].set(a[D:, 0])

    h, s12t = _phase_a(input, W, A2)

    src = edge_list[0]
    dst = edge_list[1]
    s1 = s12t[0]
    s2 = s12t[1]
    u0, u1, d0, d1 = _phase_b(src, dst, s1, s2, h)

    return _phase_c(u0, u1, d0, d1)


# speedup vs baseline: 1.0071x; 1.0071x over previous
import functools

import jax
import jax.numpy as jnp
from jax import lax
from jax.experimental import pallas as pl
from jax.experimental.pallas import tpu as pltpu
from jax.experimental.pallas import tpu_sc as plsc

N = 10000
E = 320000
D = 128
NP = 10240
RPT = NP // 16
NW = 32
EPW = E // NW
K = 80
NCHUNK = EPW // K


def _mm_body(x_ref, w_ref, a2_ref, h_ref, s_ref):
    h = jnp.dot(x_ref[...], w_ref[...], preferred_element_type=jnp.float32)
    h_ref[...] = h
    s_ref[...] = lax.dot_general(
        a2_ref[...], h, (((0,), (1,)), ((), ())),
        preferred_element_type=jnp.float32,
    )


def _phase_a(x, W, A2):
    BN = 512
    grid = NP // BN
    return pl.pallas_call(
        _mm_body,
        grid=(grid,),
        in_specs=[
            pl.BlockSpec((BN, D), lambda i: (i, 0)),
            pl.BlockSpec((D, D), lambda i: (0, 0)),
            pl.BlockSpec((D, D), lambda i: (0, 0)),
        ],
        out_specs=[
            pl.BlockSpec((BN, D), lambda i: (i, 0)),
            pl.BlockSpec((D, BN), lambda i: (0, i)),
        ],
        out_shape=[
            jax.ShapeDtypeStruct((N, D), jnp.float32),
            jax.ShapeDtypeStruct((D, NP), jnp.float32),
        ],
    )(x, W, A2)


_MESH = plsc.VectorSubcoreMesh(
    core_axis_name="c", subcore_axis_name="s", num_cores=2, num_subcores=16
)


@functools.partial(
    pl.kernel,
    out_type=[
        jax.ShapeDtypeStruct((NP, D), jnp.float32),
        jax.ShapeDtypeStruct((NP, D), jnp.float32),
        jax.ShapeDtypeStruct((NP,), jnp.float32),
        jax.ShapeDtypeStruct((NP,), jnp.float32),
    ],
    mesh=_MESH,
    compiler_params=pltpu.CompilerParams(
        needs_layout_passes=False, use_tc_tiling_on_sc=False
    ),
    scratch_types=[
        [pltpu.VMEM((4 * K,), jnp.int32)] * 2,
        [pltpu.VMEM((4 * K,), jnp.int32)] * 2,
        [pltpu.VMEM((K,), jnp.float32)] * 4,
        [pltpu.VMEM((K,), jnp.float32)] * 4,
        [pltpu.VMEM((K,), jnp.int32)] * 4,
        [pltpu.VMEM((K,), jnp.float32)] * 4,
        [pltpu.VMEM((K, D), jnp.float32)] * 4,
        pltpu.VMEM_SHARED((NP, D), jnp.float32),
        pltpu.VMEM_SHARED((NP,), jnp.float32),
        pltpu.SemaphoreType.DMA,
        pltpu.SemaphoreType.DMA,
        pltpu.SemaphoreType.DMA,
    ],
)
def _phase_b(src_hbm, dst_hbm, s1_hbm, s2_hbm, h_hbm,
             u0, u1, den0, den1,
             qsrc, qdst, s1g, s2g, scb, pbuf, rows, acc, dacc,
             gsem, isem, ssem):
    cid = lax.axis_index("c")
    sid = lax.axis_index("s")
    wid = sid * 2 + cid
    r0 = sid * RPT
    base0 = wid * EPW
    zf = jnp.zeros((16,), jnp.float32)

    def _zrow(i, _):
        for j in range(D // 16):
            rows[0][i, pl.ds(j * 16, 16)] = zf
        return 0
    lax.fori_loop(0, K, _zrow, 0)
    for i in range(K // 16):
        pbuf[0][pl.ds(i * 16, 16)] = zf
    for c in range(RPT // K):
        pltpu.async_copy(rows[0], acc.at[pl.ds(r0 + c * K, K)], ssem)
        pltpu.async_copy(pbuf[0], dacc.at[pl.ds(r0 + c * K, K)], ssem)
    for c in range(RPT // K):
        pltpu.make_async_copy(rows[0], acc.at[pl.ds(r0 + c * K, K)], ssem).wait()
        pltpu.make_async_copy(pbuf[0], dacc.at[pl.ds(r0 + c * K, K)], ssem).wait()
    plsc.subcore_barrier()

    QB = 4 * K

    def _quad_start(G, slot, n):
        e0 = base0 + G * QB
        pltpu.async_copy(src_hbm.at[pl.ds(e0, n)], qsrc[slot].at[pl.ds(0, n)],
                         isem)
        pltpu.async_copy(dst_hbm.at[pl.ds(e0, n)], qdst[slot].at[pl.ds(0, n)],
                         isem)

    def _quad_wait(slot, n):
        pltpu.make_async_copy(src_hbm.at[pl.ds(0, n)],
                              qsrc[slot].at[pl.ds(0, n)], isem).wait()
        pltpu.make_async_copy(dst_hbm.at[pl.ds(0, n)],
                              qdst[slot].at[pl.ds(0, n)], isem).wait()

    def _gather_start(q, pos, r):
        pltpu.async_copy(h_hbm.at[qdst[q].at[pl.ds(pos * K, K)]], rows[r],
                         gsem)
        pltpu.async_copy(s1_hbm.at[qsrc[q].at[pl.ds(pos * K, K)]], s1g[r],
                         gsem)
        pltpu.async_copy(s2_hbm.at[qdst[q].at[pl.ds(pos * K, K)]], s2g[r],
                         gsem)

    def _gather_wait(r):
        pltpu.make_async_copy(h_hbm.at[qdst[0].at[pl.ds(0, K)]], rows[r],
                              gsem).wait()
        pltpu.make_async_copy(s1_hbm.at[qsrc[0].at[pl.ds(0, K)]], s1g[r],
                              gsem).wait()
        pltpu.make_async_copy(s2_hbm.at[qdst[0].at[pl.ds(0, K)]], s2g[r],
                              gsem).wait()

    def _scatter_start(r):
        pltpu.async_copy(rows[r], acc.at[scb[r]], ssem, add=True)
        pltpu.async_copy(pbuf[r], dacc.at[scb[r]], ssem, add=True)

    def _scatter_wait(r):
        pltpu.make_async_copy(rows[r], acc.at[scb[r]], ssem).wait()
        pltpu.make_async_copy(pbuf[r], dacc.at[scb[r]], ssem).wait()

    def _p_loop(q, pos, r):
        for i in range(K // 16):
            isrc = qsrc[q][pl.ds(pos * K + i * 16, 16)]
            scb[r][pl.ds(i * 16, 16)] = isrc
            v = s1g[r][pl.ds(i * 16, 16)] + s2g[r][pl.ds(i * 16, 16)]
            e = jnp.where(v > 0, v, 0.2 * v)
            pbuf[r][pl.ds(i * 16, 16)] = jnp.exp(e)

    def _scale(r):
        def body(i2, _):
            for v in range(2):
                i = i2 * 2 + v
                bc = plsc.load_gather(pbuf[r], [lax.broadcast(i, (16,))])
                for j in range(D // 16):
                    rows[r][i, pl.ds(j * 16, 16)] = (
                        rows[r][i, pl.ds(j * 16, 16)] * bc
                    )
            return 0
        lax.fori_loop(0, K // 2, body, 0)

    pltpu.sync_copy(src_hbm.at[pl.ds(base0, QB)], qsrc[0])
    pltpu.sync_copy(dst_hbm.at[pl.ds(base0, QB)], qdst[0])
    _gather_start(0, 0, 0)
    _gather_start(0, 1, 1)
    _quad_start(1, 1, QB)

    def _iter(G, q, c, uu, last_quad):
        t = 4 * G + c
        _gather_wait(c)

        @pl.when(t >= 2)
        def _():
            _scatter_wait((c + 2) % 4)

        if c == 2 and not last_quad:
            _quad_wait(1 - q, QB)
        if c == 2 and last_quad:
            _quad_wait(1 - q, K)

        if not (last_quad and c == 3):
            if c <= 1:
                _gather_start(q, c + 2, (c + 2) % 4)
            else:
                _gather_start(1 - q, c - 2, (c + 2) % 4)

        _p_loop(q, c, c)

        if c == 3 and uu is not None:
            if q == 0:
                _quad_start(2 * uu + 2, 0, QB)
            else:
                @pl.when(uu < 14)
                def _():
                    _quad_start(2 * uu + 3, 1, QB)

                @pl.when(uu == 14)
                def _():
                    _quad_start(2 * uu + 3, 1, K)

        _scale(c)
        _scatter_start(c)

    def _outer(uu, _):
        for P in (0, 1):
            for c in range(4):
                _iter(2 * uu + P, P, c, uu, False)
        return 0
    lax.fori_loop(0, 15, _outer, 0)

    for c in range(4):
        _iter(30, 0, c, None, True)

    _gather_wait(0)
    _scatter_wait(2)
    _p_loop(1, 0, 0)
    _scale(0)
    _scatter_start(0)
    _scatter_wait(3)
    _scatter_wait(0)
    plsc.subcore_barrier()

    @pl.when(cid == 0)
    def _():
        pltpu.sync_copy(acc.at[pl.ds(r0, RPT)], u0.at[pl.ds(r0, RPT)])
        pltpu.sync_copy(dacc.at[pl.ds(r0, RPT)], den0.at[pl.ds(r0, RPT)])

    @pl.when(cid == 1)
    def _():
        pltpu.sync_copy(acc.at[pl.ds(r0, RPT)], u1.at[pl.ds(r0, RPT)])
        pltpu.sync_copy(dacc.at[pl.ds(r0, RPT)], den1.at[pl.ds(r0, RPT)])


def _fin_body(u0_ref, u1_ref, d0_ref, d1_ref, o_ref):
    u = u0_ref[...] + u1_ref[...]
    d = d0_ref[...] + d1_ref[...]
    r = jnp.where(d > 0, 1.0 / jnp.where(d > 0, d, 1.0), 0.0)
    hp = u * r[:, None]
    o_ref[...] = jnp.where(hp > 0, hp, jnp.exp(jnp.minimum(hp, 0.0)) - 1.0)


def _phase_c(u0, u1, d0, d1):
    BN = 512
    grid = NP // BN
    return pl.pallas_call(
        _fin_body,
        grid=(grid,),
        in_specs=[
            pl.BlockSpec((BN, D), lambda i: (i, 0)),
            pl.BlockSpec((BN, D), lambda i: (i, 0)),
            pl.BlockSpec((BN,), lambda i: (i,)),
            pl.BlockSpec((BN,), lambda i: (i,)),
        ],
        out_specs=pl.BlockSpec((BN, D), lambda i: (i, 0)),
        out_shape=jax.ShapeDtypeStruct((N, D), jnp.float32),
    )(u0, u1, d0, d1)


def kernel(input, edge_list, W, a):
    A2 = jnp.zeros((D, D), jnp.float32)
    A2 = A2.at[:, 0].set(a[:D, 0]).at[:, 1].set(a[D:, 0])

    h, s12t = _phase_a(input, W, A2)

    src = edge_list[0]
    dst = edge_list[1]
    s1 = s12t[0]
    s2 = s12t[1]
    u0, u1, d0, d1 = _phase_b(src, dst, s1, s2, h)

    return _phase_c(u0, u1, d0, d1)


# speedup vs baseline: 1.0085x; 1.0013x over previous
import functools

import jax
import jax.numpy as jnp
from jax import lax
from jax.experimental import pallas as pl
from jax.experimental.pallas import tpu as pltpu
from jax.experimental.pallas import tpu_sc as plsc

N = 10000
E = 320000
D = 128
NP = 10240
RPT = NP // 16
NW = 32
EPW = E // NW
K = 80
NCHUNK = EPW // K


def _mm_body(x_ref, w_ref, a2_ref, h_ref, s_ref):
    h = jnp.dot(x_ref[...], w_ref[...], preferred_element_type=jnp.float32)
    h_ref[...] = h
    s_ref[...] = lax.dot_general(
        a2_ref[...], h, (((0,), (1,)), ((), ())),
        preferred_element_type=jnp.float32,
    )


def _phase_a(x, W, A2):
    BN = 512
    grid = NP // BN
    return pl.pallas_call(
        _mm_body,
        grid=(grid,),
        in_specs=[
            pl.BlockSpec((BN, D), lambda i: (i, 0)),
            pl.BlockSpec((D, D), lambda i: (0, 0)),
            pl.BlockSpec((D, D), lambda i: (0, 0)),
        ],
        out_specs=[
            pl.BlockSpec((BN, D), lambda i: (i, 0)),
            pl.BlockSpec((D, BN), lambda i: (0, i)),
        ],
        out_shape=[
            jax.ShapeDtypeStruct((N, D), jnp.float32),
            jax.ShapeDtypeStruct((D, NP), jnp.float32),
        ],
    )(x, W, A2)


_MESH = plsc.VectorSubcoreMesh(
    core_axis_name="c", subcore_axis_name="s", num_cores=2, num_subcores=16
)


@functools.partial(
    pl.kernel,
    out_type=[
        jax.ShapeDtypeStruct((NP, D), jnp.float32),
        jax.ShapeDtypeStruct((NP, D), jnp.float32),
        jax.ShapeDtypeStruct((NP,), jnp.float32),
        jax.ShapeDtypeStruct((NP,), jnp.float32),
    ],
    mesh=_MESH,
    compiler_params=pltpu.CompilerParams(
        needs_layout_passes=False, use_tc_tiling_on_sc=False
    ),
    scratch_types=[
        [pltpu.VMEM((4 * K,), jnp.int32)] * 2,
        [pltpu.VMEM((4 * K,), jnp.int32)] * 2,
        [pltpu.VMEM((K,), jnp.float32)] * 4,
        [pltpu.VMEM((K,), jnp.float32)] * 4,
        [pltpu.VMEM((K,), jnp.int32)] * 4,
        [pltpu.VMEM((K,), jnp.float32)] * 4,
        [pltpu.VMEM((K, D), jnp.float32)] * 4,
        pltpu.VMEM_SHARED((NP, D), jnp.float32),
        pltpu.VMEM_SHARED((NP,), jnp.float32),
        pltpu.SemaphoreType.DMA,
        pltpu.SemaphoreType.DMA,
        pltpu.SemaphoreType.DMA,
    ],
)
def _phase_b(src_hbm, dst_hbm, s1_hbm, s2_hbm, h_hbm,
             u0, u1, den0, den1,
             qsrc, qdst, s1g, s2g, scb, pbuf, rows, acc, dacc,
             gsem, isem, ssem):
    cid = lax.axis_index("c")
    sid = lax.axis_index("s")
    wid = sid * 2 + cid
    r0 = sid * RPT
    base0 = wid * EPW
    zf = jnp.zeros((16,), jnp.float32)

    def _zrow(i, _):
        for j in range(D // 16):
            rows[0][i, pl.ds(j * 16, 16)] = zf
        return 0
    lax.fori_loop(0, K, _zrow, 0)
    for i in range(K // 16):
        pbuf[0][pl.ds(i * 16, 16)] = zf
    for c in range(RPT // K):
        pltpu.async_copy(rows[0], acc.at[pl.ds(r0 + c * K, K)], ssem)
        pltpu.async_copy(pbuf[0], dacc.at[pl.ds(r0 + c * K, K)], ssem)
    for c in range(RPT // K):
        pltpu.make_async_copy(rows[0], acc.at[pl.ds(r0 + c * K, K)], ssem).wait()
        pltpu.make_async_copy(pbuf[0], dacc.at[pl.ds(r0 + c * K, K)], ssem).wait()
    plsc.subcore_barrier()

    QB = 4 * K

    def _quad_start(G, slot, n):
        e0 = base0 + G * QB
        pltpu.async_copy(src_hbm.at[pl.ds(e0, n)], qsrc[slot].at[pl.ds(0, n)],
                         isem)
        pltpu.async_copy(dst_hbm.at[pl.ds(e0, n)], qdst[slot].at[pl.ds(0, n)],
                         isem)

    def _quad_wait(slot, n):
        pltpu.make_async_copy(src_hbm.at[pl.ds(0, n)],
                              qsrc[slot].at[pl.ds(0, n)], isem).wait()
        pltpu.make_async_copy(dst_hbm.at[pl.ds(0, n)],
                              qdst[slot].at[pl.ds(0, n)], isem).wait()

    def _gather_start(q, pos, r):
        pltpu.async_copy(h_hbm.at[qdst[q].at[pl.ds(pos * K, K)]], rows[r],
                         gsem)
        pltpu.async_copy(s1_hbm.at[qsrc[q].at[pl.ds(pos * K, K)]], s1g[r],
                         gsem)
        pltpu.async_copy(s2_hbm.at[qdst[q].at[pl.ds(pos * K, K)]], s2g[r],
                         gsem)

    def _gather_wait(r):
        pltpu.make_async_copy(h_hbm.at[qdst[0].at[pl.ds(0, K)]], rows[r],
                              gsem).wait()
        pltpu.make_async_copy(s1_hbm.at[qsrc[0].at[pl.ds(0, K)]], s1g[r],
                              gsem).wait()
        pltpu.make_async_copy(s2_hbm.at[qdst[0].at[pl.ds(0, K)]], s2g[r],
                              gsem).wait()

    def _scatter_start(r):
        pltpu.async_copy(rows[r], acc.at[scb[r]], ssem, add=True)
        pltpu.async_copy(pbuf[r], dacc.at[scb[r]], ssem, add=True)

    def _scatter_wait(r):
        pltpu.make_async_copy(rows[r], acc.at[scb[r]], ssem).wait()
        pltpu.make_async_copy(pbuf[r], dacc.at[scb[r]], ssem).wait()

    def _p_loop(q, pos, r):
        for i in range(K // 16):
            isrc = qsrc[q][pl.ds(pos * K + i * 16, 16)]
            scb[r][pl.ds(i * 16, 16)] = isrc
            v = s1g[r][pl.ds(i * 16, 16)] + s2g[r][pl.ds(i * 16, 16)]
            e = jnp.where(v > 0, v, 0.2 * v)
            pbuf[r][pl.ds(i * 16, 16)] = jnp.exp(e)

    def _scale(r):
        def body(i2, _):
            for v in range(2):
                i = i2 * 2 + v
                bc = plsc.load_gather(pbuf[r], [lax.broadcast(i, (16,))])
                for j in range(D // 16):
                    rows[r][i, pl.ds(j * 16, 16)] = (
                        rows[r][i, pl.ds(j * 16, 16)] * bc
                    )
            return 0
        lax.fori_loop(0, K // 2, body, 0)

    pltpu.sync_copy(src_hbm.at[pl.ds(base0, QB)], qsrc[0])
    pltpu.sync_copy(dst_hbm.at[pl.ds(base0, QB)], qdst[0])
    _gather_start(0, 0, 0)
    _gather_start(0, 1, 1)
    _quad_start(1, 1, QB)

    def _iter(G, q, c, uu, last_quad):
        t = 4 * G + c
        _gather_wait(c)
        _p_loop(q, c, c)

        @pl.when(t >= 2)
        def _():
            _scatter_wait((c + 2) % 4)

        if c == 2 and not last_quad:
            _quad_wait(1 - q, QB)
        if c == 2 and last_quad:
            _quad_wait(1 - q, K)

        if not (last_quad and c == 3):
            if c <= 1:
                _gather_start(q, c + 2, (c + 2) % 4)
            else:
                _gather_start(1 - q, c - 2, (c + 2) % 4)

        if c == 3 and uu is not None:
            if q == 0:
                _quad_start(2 * uu + 2, 0, QB)
            else:
                @pl.when(uu < 14)
                def _():
                    _quad_start(2 * uu + 3, 1, QB)

                @pl.when(uu == 14)
                def _():
                    _quad_start(2 * uu + 3, 1, K)

        _scale(c)
        _scatter_start(c)

    def _outer(uu, _):
        for P in (0, 1):
            for c in range(4):
                _iter(2 * uu + P, P, c, uu, False)
        return 0
    lax.fori_loop(0, 15, _outer, 0)

    for c in range(4):
        _iter(30, 0, c, None, True)

    _gather_wait(0)
    _scatter_wait(2)
    _p_loop(1, 0, 0)
    _scale(0)
    _scatter_start(0)
    _scatter_wait(3)
    _scatter_wait(0)
    plsc.subcore_barrier()

    @pl.when(cid == 0)
    def _():
        pltpu.sync_copy(acc.at[pl.ds(r0, RPT)], u0.at[pl.ds(r0, RPT)])
        pltpu.sync_copy(dacc.at[pl.ds(r0, RPT)], den0.at[pl.ds(r0, RPT)])

    @pl.when(cid == 1)
    def _():
        pltpu.sync_copy(acc.at[pl.ds(r0, RPT)], u1.at[pl.ds(r0, RPT)])
        pltpu.sync_copy(dacc.at[pl.ds(r0, RPT)], den1.at[pl.ds(r0, RPT)])


def _fin_body(u0_ref, u1_ref, d0_ref, d1_ref, o_ref):
    u = u0_ref[...] + u1_ref[...]
    d = d0_ref[...] + d1_ref[...]
    r = jnp.where(d > 0, 1.0 / jnp.where(d > 0, d, 1.0), 0.0)
    hp = u * r[:, None]
    o_ref[...] = jnp.where(hp > 0, hp, jnp.exp(jnp.minimum(hp, 0.0)) - 1.0)


def _phase_c(u0, u1, d0, d1):
    BN = 512
    grid = NP // BN
    return pl.pallas_call(
        _fin_body,
        grid=(grid,),
        in_specs=[
            pl.BlockSpec((BN, D), lambda i: (i, 0)),
            pl.BlockSpec((BN, D), lambda i: (i, 0)),
            pl.BlockSpec((BN,), lambda i: (i,)),
            pl.BlockSpec((BN,), lambda i: (i,)),
        ],
        out_specs=pl.BlockSpec((BN, D), lambda i: (i, 0)),
        out_shape=jax.ShapeDtypeStruct((N, D), jnp.float32),
    )(u0, u1, d0, d1)


def kernel(input, edge_list, W, a):
    A2 = jnp.zeros((D, D), jnp.float32)
    A2 = A2.at[:, 0].set(a[:D, 0]).at[:, 1].set(a[D:, 0])

    h, s12t = _phase_a(input, W, A2)

    src = edge_list[0]
    dst = edge_list[1]
    s1 = s12t[0]
    s2 = s12t[1]
    u0, u1, d0, d1 = _phase_b(src, dst, s1, s2, h)

    return _phase_c(u0, u1, d0, d1)
